# Initial kernel scaffold; baseline (speedup 1.0000x reference)
import jax
import jax.numpy as jnp
from jax.experimental import pallas as pl


def kernel(x_nchw, w1, w2, w3, w4, w5, g1, b1, g2, b2, g3, b3, g4, b4):
    raise NotImplementedError("write your pallas kernel here")


# speedup vs baseline: 10.6743x; 10.6743x over previous
import jax
import jax.numpy as jnp
from jax import lax
from jax.experimental import pallas as pl
from jax.experimental.pallas import tpu as pltpu

_EPS = 1e-5
_N = 16

_TAPMAP = {0: (3, 1), 1: (2, 0)}


def _bn_relu(ys, g, b, m_real):
    c = ys[0].shape[-1]
    s = jnp.zeros((1, c), jnp.float32)
    sq = jnp.zeros((1, c), jnp.float32)
    for y in ys:
        s = s + jnp.sum(y, axis=0, keepdims=True)
        sq = sq + jnp.sum(y * y, axis=0, keepdims=True)
    inv_m = 1.0 / m_real
    mean = s * inv_m
    var = sq * inv_m - mean * mean
    scale = g * lax.rsqrt(var + _EPS)
    shift = b - mean * scale
    return [jnp.maximum(y * scale + shift, 0.0) for y in ys]


def _pad_hw(x):
    h, w, n, c = x.shape
    zr = jnp.zeros((1, w, n, c), x.dtype)
    x = jnp.concatenate([zr, x, zr], axis=0)
    zc = jnp.zeros((h + 2, 1, n, c), x.dtype)
    return jnp.concatenate([zc, x, zc], axis=1)


def _parity_patches(xp, ph, pw, h, w):
    c = xp.shape[-1]
    taps = [xp[ph + dh:ph + dh + h, pw + dw:pw + dw + w].reshape(h * w * _N, c)
            for dh in (0, 1) for dw in (0, 1)]
    return jnp.concatenate(taps, axis=-1)


def _gen_kernel(x_ref, w1_ref, w2_ref, w3_ref, w4_ref, w5_ref,
                g1_ref, b1_ref, g2_ref, b2_ref, g3_ref, b3_ref, g4_ref, b4_ref,
                o_ref):
    x0 = x_ref[...]

    ys = [jnp.dot(x0, w1_ref[i], preferred_element_type=jnp.float32)
          for i in range(16)]
    ys = _bn_relu(ys, g1_ref[...], b1_ref[...], float(_N * 16))
    x = jnp.stack(ys, axis=0).reshape(4, 4, _N, 512).astype(jnp.bfloat16)

    for w_ref, g_ref, b_ref, (h, w, co) in (
            (w2_ref, g2_ref, b2_ref, (4, 4, 256)),
            (w3_ref, g3_ref, b3_ref, (8, 8, 128)),
            (w4_ref, g4_ref, b4_ref, (16, 16, 64))):
        xp = _pad_hw(x)
        yps = []
        for ph in (0, 1):
            for pw in (0, 1):
                a = _parity_patches(xp, ph, pw, h, w)
                yps.append(jnp.dot(a, w_ref[2 * ph + pw],
                                   preferred_element_type=jnp.float32))
        yps = _bn_relu(yps, g_ref[...], b_ref[...], float(4 * h * w * _N))
        t = [y.reshape(h, w, _N, co) for y in yps]
        top = jnp.stack([t[0], t[1]], axis=2).reshape(h, 2 * w, _N, co)
        bot = jnp.stack([t[2], t[3]], axis=2).reshape(h, 2 * w, _N, co)
        x = (jnp.stack([top, bot], axis=1)
             .reshape(2 * h, 2 * w, _N, co).astype(jnp.bfloat16))

    xp = _pad_hw(x)
    for ph in (0, 1):
        for pw in (0, 1):
            a = _parity_patches(xp, ph, pw, 32, 32)
            y = jnp.dot(a, w5_ref[2 * ph + pw],
                        preferred_element_type=jnp.float32)
            o_ref[2 * ph + pw] = jnp.tanh(y)


def _prep_s2_weights(w, cpad=None):
    cin, cout = w.shape[0], w.shape[1]
    bs = []
    for ph in (0, 1):
        for pw in (0, 1):
            wsub = jnp.stack(
                [jnp.stack([w[:, :, _TAPMAP[ph][dh], _TAPMAP[pw][dw]]
                            for dw in (0, 1)], axis=0)
                 for dh in (0, 1)], axis=0)
            bs.append(wsub.reshape(4 * cin, cout))
    b = jnp.stack(bs, axis=0)
    if cpad is not None and cpad != cout:
        b = jnp.pad(b, ((0, 0), (0, 0), (0, cpad - cout)))
    return b.astype(jnp.bfloat16)


def kernel(x_nchw, w1, w2, w3, w4, w5, g1, b1, g2, b2, g3, b3, g4, b4):
    x0 = x_nchw.reshape(_N, 256).astype(jnp.bfloat16)
    w1p = jnp.transpose(w1, (2, 3, 0, 1)).reshape(16, 256, 512).astype(jnp.bfloat16)
    w2p = _prep_s2_weights(w2)
    w3p = _prep_s2_weights(w3)
    w4p = _prep_s2_weights(w4)
    w5p = _prep_s2_weights(w5, cpad=8)

    def v(a):
        return a.reshape(1, -1).astype(jnp.float32)

    y = pl.pallas_call(
        _gen_kernel,
        out_shape=jax.ShapeDtypeStruct((4, 32 * 32 * _N, 8), jnp.float32),
        compiler_params=pltpu.CompilerParams(
            vmem_limit_bytes=56 * 1024 * 1024),
    )(x0, w1p, w2p, w3p, w4p, w5p,
      v(g1), v(b1), v(g2), v(b2), v(g3), v(b3), v(g4), v(b4))

    img = y[:, :, 0].reshape(2, 2, 32, 32, _N)
    img = jnp.transpose(img, (4, 2, 0, 3, 1)).reshape(_N, 64, 64)
    return img[:, None, :, :]


# speedup vs baseline: 11.5779x; 1.0847x over previous
import jax
import jax.numpy as jnp
from jax import lax
from jax.experimental import pallas as pl
from jax.experimental.pallas import tpu as pltpu

_EPS = 1e-5
_N = 16

_TAPMAP = {0: (3, 1), 1: (2, 0)}


def _bn_relu(ys, g, b, m_real):
    c = ys[0].shape[-1]
    s = jnp.zeros((1, c), jnp.float32)
    sq = jnp.zeros((1, c), jnp.float32)
    for y in ys:
        s = s + jnp.sum(y, axis=0, keepdims=True)
        sq = sq + jnp.sum(y * y, axis=0, keepdims=True)
    inv_m = 1.0 / m_real
    mean = s * inv_m
    var = sq * inv_m - mean * mean
    scale = g * lax.rsqrt(var + _EPS)
    shift = b - mean * scale
    return [jnp.maximum(y * scale + shift, 0.0) for y in ys]


def _pad_hw(x):
    h, w, n, c = x.shape
    zr = jnp.zeros((1, w, n, c), x.dtype)
    x = jnp.concatenate([zr, x, zr], axis=0)
    zc = jnp.zeros((h + 2, 1, n, c), x.dtype)
    return jnp.concatenate([zc, x, zc], axis=1)


def _parity_patches(xp, ph, pw, h, w):
    c = xp.shape[-1]
    taps = [xp[ph + dh:ph + dh + h, pw + dw:pw + dw + w].reshape(h * w * _N, c)
            for dh in (0, 1) for dw in (0, 1)]
    return jnp.concatenate(taps, axis=-1)


def _gen_kernel(x_ref, w1_ref, w2_ref, w3_ref, w4_ref, w5_ref,
                g1_ref, b1_ref, g2_ref, b2_ref, g3_ref, b3_ref, g4_ref, b4_ref,
                o_ref):
    x0 = x_ref[...].astype(jnp.bfloat16)

    ys = [jnp.dot(x0, w1_ref[i], preferred_element_type=jnp.float32)
          for i in range(16)]
    ys = _bn_relu(ys, g1_ref[...], b1_ref[...], float(_N * 16))
    x = jnp.stack(ys, axis=0).reshape(4, 4, _N, 512).astype(jnp.bfloat16)

    for w_ref, g_ref, b_ref, (h, w, co) in (
            (w2_ref, g2_ref, b2_ref, (4, 4, 256)),
            (w3_ref, g3_ref, b3_ref, (8, 8, 128)),
            (w4_ref, g4_ref, b4_ref, (16, 16, 64))):
        xp = _pad_hw(x)
        yps = []
        for ph in (0, 1):
            for pw in (0, 1):
                a = _parity_patches(xp, ph, pw, h, w)
                yps.append(jnp.dot(a, w_ref[2 * ph + pw],
                                   preferred_element_type=jnp.float32))
        yps = _bn_relu(yps, g_ref[...], b_ref[...], float(4 * h * w * _N))
        t = [y.reshape(h, w, _N, co) for y in yps]
        top = jnp.stack([t[0], t[1]], axis=2).reshape(h, 2 * w, _N, co)
        bot = jnp.stack([t[2], t[3]], axis=2).reshape(h, 2 * w, _N, co)
        x = (jnp.stack([top, bot], axis=1)
             .reshape(2 * h, 2 * w, _N, co).astype(jnp.bfloat16))

    xp = _pad_hw(x)
    for ph in (0, 1):
        for pw in (0, 1):
            a = _parity_patches(xp, ph, pw, 32, 32)
            y = jnp.dot(a, w5_ref[2 * ph + pw],
                        preferred_element_type=jnp.float32)
            o_ref[2 * ph + pw] = jnp.tanh(y)


def _prep_s2_weights(w, cpad=None):
    bs = []
    for ph in (0, 1):
        for pw in (0, 1):
            bs.append(jnp.concatenate(
                [w[:, :, _TAPMAP[ph][dh], _TAPMAP[pw][dw]]
                 for dh in (0, 1) for dw in (0, 1)], axis=0))
    b = jnp.stack(bs, axis=0)
    if cpad is not None and cpad != b.shape[-1]:
        b = jnp.pad(b, ((0, 0), (0, 0), (0, cpad - b.shape[-1])))
    return b.astype(jnp.bfloat16)


def kernel(x_nchw, w1, w2, w3, w4, w5, g1, b1, g2, b2, g3, b3, g4, b4):
    x0 = x_nchw.reshape(_N, 256)
    w1p = jnp.stack([w1[:, :, i // 4, i % 4] for i in range(16)],
                    axis=0).astype(jnp.bfloat16)
    w2p = _prep_s2_weights(w2)
    w3p = _prep_s2_weights(w3)
    w4p = _prep_s2_weights(w4)
    w5p = _prep_s2_weights(w5, cpad=8)

    def v(a):
        return a.reshape(1, -1).astype(jnp.float32)

    y = pl.pallas_call(
        _gen_kernel,
        out_shape=jax.ShapeDtypeStruct((4, 32 * 32 * _N, 8), jnp.float32),
        compiler_params=pltpu.CompilerParams(
            vmem_limit_bytes=56 * 1024 * 1024),
    )(x0, w1p, w2p, w3p, w4p, w5p,
      v(g1), v(b1), v(g2), v(b2), v(g3), v(b3), v(g4), v(b4))

    img = y[:, :, 0].reshape(2, 2, 32, 32, _N)
    img = jnp.transpose(img, (4, 2, 0, 3, 1)).reshape(_N, 64, 64)
    return img[:, None, :, :]
